# baseline (speedup 1.0000x reference)
import functools

import jax
import jax.numpy as jnp
from jax import lax
from jax.experimental import pallas as pl
from jax.experimental.pallas import tpu as pltpu
from jax.experimental.pallas import tpu_sc as plsc

_NC = 2
_NS = 16
_NW = _NC * _NS
_L = 16


def _make_sc_gather(R, H, W, C):
    rpw = R // _NW
    sel_len = 128
    n_chunks = (C + _L - 1) // _L

    mesh = plsc.VectorSubcoreMesh(core_axis_name="c", subcore_axis_name="s")

    @functools.partial(
        pl.kernel,
        mesh=mesh,
        out_type=jax.ShapeDtypeStruct((R, H, W), jnp.float32),
        compiler_params=pltpu.CompilerParams(needs_layout_passes=False),
        scratch_types=[
            pltpu.VMEM((C,), jnp.float32),
            pltpu.VMEM((sel_len,), jnp.int32),
            pltpu.VMEM((16 * _L,), jnp.int32),
            pltpu.VMEM((1, H // 2, W), jnp.float32),
            pltpu.VMEM((1, H // 2, W), jnp.float32),
            pltpu.VMEM((1, H // 2, W), jnp.float32),
            pltpu.VMEM((1, H // 2, W), jnp.float32),
            pltpu.SemaphoreType.DMA,
            pltpu.SemaphoreType.DMA,
            pltpu.SemaphoreType.DMA,
            pltpu.SemaphoreType.DMA,
            pltpu.SemaphoreType.DMA,
            pltpu.SemaphoreType.DMA,
            pltpu.SemaphoreType.DMA,
            pltpu.SemaphoreType.DMA,
        ],
    )
    def sc_gather(inp, idxs, out, mask_v, sel_v, rix_v, buf0, buf1, buf2,
                  buf3, gs0, gs1, gs2, gs3, ss0, ss1, ss2, ss3):
        wid = lax.axis_index("s") * _NC + lax.axis_index("c")
        base = wid * rpw
        n = base // C
        co = base - n * C

        pltpu.sync_copy(idxs, mask_v)

        zeros = jnp.zeros((_L,), jnp.int32)
        for t in range(sel_len // _L):
            sel_v[pl.ds(t * _L, _L)] = zeros
        carry = jnp.int32(0)
        for t in range(n_chunks):
            v = mask_v[pl.ds(t * _L, _L)]
            nz = v != 0.0
            nzi = nz.astype(jnp.int32)
            pos = plsc.cumsum(nzi) - 1 + carry
            pos = jnp.where(nz, pos, 0)
            j = lax.iota(jnp.int32, _L) + (t * _L)
            plsc.store_scatter(sel_v, [pos], j, mask=nz)
            carry = carry + jnp.sum(nzi)

        nbase = n * C
        for t in range(2):
            lane = lax.iota(jnp.int32, _L) + (co + t * _L)
            sel = plsc.load_gather(sel_v, [lane])
            slot = (lax.iota(jnp.int32, _L) + t * _L) * 8
            plsc.store_scatter(rix_v, [slot], sel + nbase)

        def src_row(i):
            return rix_v[pl.ds(8 * i, _L)][0]

        half = H // 2
        nch = 2 * rpw
        depth = 4
        bufs = (buf0, buf1, buf2, buf3)
        gsems = (gs0, gs1, gs2, gs3)
        ssems = (ss0, ss1, ss2, ss3)

        def in_chunk(j):
            return inp.at[pl.ds(src_row(j // 2), 1),
                          pl.ds((j % 2) * half, half)]

        def out_chunk(j):
            return out.at[pl.ds(base + j // 2, 1),
                          pl.ds((j % 2) * half, half)]

        gather_dma = [None] * depth
        scatter_dma = [None] * depth
        for j in range(depth):
            gather_dma[j] = pltpu.async_copy(in_chunk(j), bufs[j], gsems[j])
        for j in range(nch):
            b = j % depth
            gather_dma[b].wait()
            scatter_dma[b] = pltpu.async_copy(bufs[b], out_chunk(j), ssems[b])
            if j + depth < nch:
                scatter_dma[b].wait()
                gather_dma[b] = pltpu.async_copy(
                    in_chunk(j + depth), bufs[b], gsems[b])
        for b in range(depth):
            scatter_dma[b].wait()

    return sc_gather


def kernel(input_tensor, indexes):
    N, C, H, W = input_tensor.shape
    R = N * C
    inp3d = input_tensor.reshape(R, H, W)
    out3d = _make_sc_gather(R, H, W, C)(inp3d, indexes)
    return out3d.reshape(N, C, H, W)


# speedup vs baseline: 1.0783x; 1.0783x over previous
import functools

import jax
import jax.numpy as jnp
from jax import lax
from jax.experimental import pallas as pl
from jax.experimental.pallas import tpu as pltpu
from jax.experimental.pallas import tpu_sc as plsc

_NC = 2
_NS = 16
_NW = _NC * _NS
_L = 16


def _make_sc_gather(R, H, W, C):
    rpw = R // _NW
    sel_len = 128
    n_chunks = (C + _L - 1) // _L

    mesh = plsc.VectorSubcoreMesh(core_axis_name="c", subcore_axis_name="s")

    @functools.partial(
        pl.kernel,
        mesh=mesh,
        out_type=jax.ShapeDtypeStruct((R, H, W), jnp.float32),
        compiler_params=pltpu.CompilerParams(needs_layout_passes=False),
        scratch_types=[
            pltpu.VMEM((C,), jnp.float32),
            pltpu.VMEM((sel_len,), jnp.int32),
            pltpu.VMEM((16 * _L,), jnp.int32),
            pltpu.VMEM_SHARED((_NS, 2, 1, H, W), jnp.float32),
            pltpu.SemaphoreType.DMA,
            pltpu.SemaphoreType.DMA,
            pltpu.SemaphoreType.DMA,
            pltpu.SemaphoreType.DMA,
        ],
    )
    def sc_gather(inp, idxs, out, mask_v, sel_v, rix_v, shbuf,
                  gs0, gs1, ss0, ss1):
        wid = lax.axis_index("s") * _NC + lax.axis_index("c")
        base = wid * rpw
        n = base // C
        co = base - n * C

        pltpu.sync_copy(idxs, mask_v)

        zeros = jnp.zeros((_L,), jnp.int32)
        for t in range(sel_len // _L):
            sel_v[pl.ds(t * _L, _L)] = zeros
        carry = jnp.int32(0)
        for t in range(n_chunks):
            v = mask_v[pl.ds(t * _L, _L)]
            nz = v != 0.0
            nzi = nz.astype(jnp.int32)
            pos = plsc.cumsum(nzi) - 1 + carry
            pos = jnp.where(nz, pos, 0)
            j = lax.iota(jnp.int32, _L) + (t * _L)
            plsc.store_scatter(sel_v, [pos], j, mask=nz)
            carry = carry + jnp.sum(nzi)

        nbase = n * C
        for t in range(2):
            lane = lax.iota(jnp.int32, _L) + (co + t * _L)
            sel = plsc.load_gather(sel_v, [lane])
            slot = (lax.iota(jnp.int32, _L) + t * _L) * 8
            plsc.store_scatter(rix_v, [slot], sel + nbase)

        def src_row(i):
            return rix_v[pl.ds(8 * i, _L)][0]

        sid = lax.axis_index("s")
        bufs = (shbuf.at[sid, 0], shbuf.at[sid, 1])
        gsems = (gs0, gs1)
        ssems = (ss0, ss1)
        gather_dma = [None, None]
        scatter_dma = [None, None]
        gather_dma[0] = pltpu.async_copy(
            inp.at[pl.ds(src_row(0), 1)], bufs[0], gsems[0])
        for i in range(rpw):
            b = i % 2
            nb = (i + 1) % 2
            if i + 1 < rpw:
                if scatter_dma[nb] is not None:
                    scatter_dma[nb].wait()
                gather_dma[nb] = pltpu.async_copy(
                    inp.at[pl.ds(src_row(i + 1), 1)], bufs[nb], gsems[nb])
            gather_dma[b].wait()
            scatter_dma[b] = pltpu.async_copy(
                bufs[b], out.at[pl.ds(base + i, 1)], ssems[b])
        scatter_dma[0].wait()
        scatter_dma[1].wait()

    return sc_gather


def kernel(input_tensor, indexes):
    N, C, H, W = input_tensor.shape
    R = N * C
    inp3d = input_tensor.reshape(R, H, W)
    out3d = _make_sc_gather(R, H, W, C)(inp3d, indexes)
    return out3d.reshape(N, C, H, W)
